# baseline (speedup 1.0000x reference)
import functools

import jax
import jax.numpy as jnp
from jax import lax
from jax.experimental import pallas as pl
from jax.experimental.pallas import tpu as pltpu
from jax.experimental.pallas import tpu_sc as plsc

DIM = 64
DPAD = 128
RELA_NUM = 3
B, L, N = 1024, 50, 16
BL = B * L
NW = 32
PW = BL // NW
P = 16
NIT = PW // P
NC = P * N // 128
NROWS = BL * N // 128


def _body(seq_hbm, nbr_hbm, rel_hbm, tab_hbm, wb_hbm, r_hbm, out_hbm,
          seq_idx_v, nbr_idx_v, rel_v, wb_v, r_v,
          seq_rows_v, nbr_rows_v, out_v, sem_in0, sem_in1, sem_out0, sem_out1):
    wid = lax.axis_index("s") * 2 + lax.axis_index("c")
    sem_in = (sem_in0, sem_in1)
    sem_out = (sem_out0, sem_out1)

    pltpu.sync_copy(wb_hbm, wb_v)
    pltpu.sync_copy(r_hbm, r_v)
    wb = wb_v[...]
    m = jnp.maximum(jnp.maximum(wb[0], wb[1]), wb[2])
    e = jnp.exp(wb - m)
    beta = e / (e[0] + e[1] + e[2])
    w_r = []
    for r in range(RELA_NUM):
        t = r_v[r] * beta
        w_r.append(t[0] + t[1] + t[2])

    def fire(it, slot):
        base = wid * PW + it * P
        rbase = wid * (PW // 8) + it * (P // 8)
        pltpu.sync_copy(seq_hbm.at[pl.ds(base, P)], seq_idx_v.at[slot])
        pltpu.sync_copy(nbr_hbm.at[pl.ds(rbase, NC)], nbr_idx_v.at[slot])
        pltpu.sync_copy(rel_hbm.at[pl.ds(rbase, NC)], rel_v.at[slot])
        pltpu.async_copy(tab_hbm.at[seq_idx_v.at[slot]],
                         seq_rows_v.at[slot], sem_in[slot])
        for c in range(NC):
            pltpu.async_copy(tab_hbm.at[nbr_idx_v.at[slot, c]],
                             nbr_rows_v.at[slot, pl.ds(c * 128, 128)],
                             sem_in[slot])

    def wait_rows(slot):
        pltpu.make_async_copy(tab_hbm.at[seq_idx_v.at[slot]],
                              seq_rows_v.at[slot], sem_in[slot]).wait()
        for c in range(NC):
            pltpu.make_async_copy(tab_hbm.at[nbr_idx_v.at[slot, c]],
                                  nbr_rows_v.at[slot, pl.ds(c * 128, 128)],
                                  sem_in[slot]).wait()

    def drain_out(slot):
        pltpu.make_async_copy(out_v.at[slot],
                              out_hbm.at[pl.ds(0, P)], sem_out[slot]).wait()

    def compute(it, slot):
        base = wid * PW + it * P
        wait_rows(slot)

        @pl.when(it >= 2)
        def _():
            drain_out(slot)

        lane = lax.iota(jnp.int32, 16)
        bidx = [lane ^ k for k in (8, 4, 2, 1)]
        nidx = [jnp.full((16,), n, jnp.int32) for n in range(N)]

        def p_body(p, c2):
            ri = p // 8
            cb = (p % 8) * 16
            accs = [seq_rows_v[slot, p, pl.ds(d * 16, 16)] for d in range(4)]
            rel_vec = rel_v[slot, ri, pl.ds(cb, 16)]
            w_vec = jnp.where(rel_vec == 0, w_r[0],
                              jnp.where(rel_vec == 1, w_r[1], w_r[2]))
            for n in range(N):
                w_n = w_vec.at[nidx[n]].get(mode="promise_in_bounds")
                row = p * N + n
                for d in range(4):
                    accs[d] = accs[d] + w_n * nbr_rows_v[slot, row,
                                                         pl.ds(d * 16, 16)]
            ssv = accs[0] * accs[0]
            for d in range(1, 4):
                ssv = ssv + accs[d] * accs[d]
            for ix in bidx:
                ssv = ssv + ssv.at[ix].get(mode="promise_in_bounds")
            x = jnp.maximum(ssv, jnp.float32(1e-24))
            yi = jnp.int32(0x5F3759DF) - lax.shift_right_logical(
                lax.bitcast_convert_type(x, jnp.int32), 1)
            y = lax.bitcast_convert_type(yi, jnp.float32)
            for _ in range(3):
                y = y * (1.5 - 0.5 * x * y * y)
            for d in range(4):
                out_v[slot, p, pl.ds(d * 16, 16)] = accs[d] * y
            return c2

        lax.fori_loop(0, P, p_body, 0)
        pltpu.async_copy(out_v.at[slot], out_hbm.at[pl.ds(base, P)],
                         sem_out[slot])

    fire(0, 0)

    def g_body(g, carry):
        it = g * 2
        fire(it + 1, 1)
        compute(it, 0)

        @pl.when(it + 2 < NIT)
        def _():
            fire(it + 2, 0)

        compute(it + 1, 1)
        return carry

    lax.fori_loop(0, NIT // 2, g_body, 0)
    drain_out(0)
    drain_out(1)


@jax.jit
def _run(seq_flat, nbr2d, rel2d, uem, wb_pad, r_pad):
    uem = jnp.pad(uem, ((0, 0), (0, DPAD - DIM)))
    mesh = plsc.VectorSubcoreMesh(core_axis_name="c", subcore_axis_name="s")
    k = functools.partial(
        pl.kernel,
        mesh=mesh,
        compiler_params=pltpu.CompilerParams(use_tc_tiling_on_sc=False),
        out_type=jax.ShapeDtypeStruct((BL, DIM), jnp.float32),
        scratch_types=[
            pltpu.VMEM((2, P), jnp.int32),
            pltpu.VMEM((2, NC, 128), jnp.int32),
            pltpu.VMEM((2, NC, 128), jnp.int32),
            pltpu.VMEM((16,), jnp.float32),
            pltpu.VMEM((RELA_NUM, 16), jnp.float32),
            pltpu.VMEM((2, P, DPAD), jnp.float32),
            pltpu.VMEM((2, P * N, DPAD), jnp.float32),
            pltpu.VMEM((2, P, DIM), jnp.float32),
            pltpu.SemaphoreType.DMA,
            pltpu.SemaphoreType.DMA,
            pltpu.SemaphoreType.DMA,
            pltpu.SemaphoreType.DMA,
        ],
    )(_body)
    return k(seq_flat, nbr2d, rel2d, uem, wb_pad, r_pad)


def kernel(sequence, seq_neighbor, rel_neigh, UEm, R, weight_b):
    seq_flat = sequence.reshape(BL)
    nbr2d = seq_neighbor.reshape(NROWS, 128)
    rel2d = rel_neigh.reshape(NROWS, 128)
    wb_pad = jnp.full((16,), -1e30, jnp.float32).at[:RELA_NUM].set(weight_b[:, 0])
    r_pad = jnp.zeros((RELA_NUM, 16), jnp.float32).at[:, :RELA_NUM].set(R)
    out = _run(seq_flat, nbr2d, rel2d, UEm, wb_pad, r_pad)
    return out.reshape(B, L, DIM)


# speedup vs baseline: 1.1130x; 1.1130x over previous
import functools

import jax
import jax.numpy as jnp
from jax import lax
from jax.experimental import pallas as pl
from jax.experimental.pallas import tpu as pltpu
from jax.experimental.pallas import tpu_sc as plsc

DIM = 64
DPAD = 128
RELA_NUM = 3
B, L, N = 1024, 50, 16
BL = B * L
NW = 32
PW = BL // NW
P = 16
NIT = PW // P
NC = P * N // 128
NROWS = BL * N // 128
WR = NROWS // NW


def _body(seq_hbm, nbr_hbm, rel_hbm, tab_hbm, wb_hbm, r_hbm, out_hbm,
          seq_idx_v, nbr_idx_v, rel_v, wb_v, r_v,
          seq_rows_v, nbr_rows_v, out_v, sem_in0, sem_in1, sem_out0, sem_out1):
    wid = lax.axis_index("s") * 2 + lax.axis_index("c")
    sem_in = (sem_in0, sem_in1)
    sem_out = (sem_out0, sem_out1)

    pltpu.sync_copy(wb_hbm, wb_v)
    pltpu.sync_copy(r_hbm, r_v)
    wb = wb_v[pl.ds(0, 16)]
    m = jnp.maximum(jnp.maximum(wb[0], wb[1]), wb[2])
    e = jnp.exp(wb - m)
    beta = e / (e[0] + e[1] + e[2])
    w_r = []
    for r in range(RELA_NUM):
        t = r_v[r, pl.ds(0, 16)] * beta
        w_r.append(t[0] + t[1] + t[2])

    pltpu.sync_copy(seq_hbm.at[pl.ds(wid * PW, PW)], seq_idx_v)
    pltpu.sync_copy(nbr_hbm.at[pl.ds(wid * WR, WR)], nbr_idx_v)
    pltpu.sync_copy(rel_hbm.at[pl.ds(wid * WR, WR)], rel_v)

    def fire(it, slot):
        pltpu.async_copy(tab_hbm.at[seq_idx_v.at[pl.ds(it * P, P)]],
                         seq_rows_v.at[slot], sem_in[slot])
        for c in range(NC):
            pltpu.async_copy(tab_hbm.at[nbr_idx_v.at[it * NC + c]],
                             nbr_rows_v.at[slot, pl.ds(c * 128, 128)],
                             sem_in[slot])

    def wait_rows(it, slot):
        pltpu.make_async_copy(tab_hbm.at[seq_idx_v.at[pl.ds(it * P, P)]],
                              seq_rows_v.at[slot], sem_in[slot]).wait()
        for c in range(NC):
            pltpu.make_async_copy(tab_hbm.at[nbr_idx_v.at[it * NC + c]],
                                  nbr_rows_v.at[slot, pl.ds(c * 128, 128)],
                                  sem_in[slot]).wait()

    def drain_out(slot):
        pltpu.make_async_copy(out_v.at[slot],
                              out_hbm.at[pl.ds(0, P)], sem_out[slot]).wait()

    def compute(it, slot):
        base = wid * PW + it * P
        wait_rows(it, slot)

        @pl.when(it >= 2)
        def _():
            drain_out(slot)

        lane = lax.iota(jnp.int32, 16)
        bidx = [lane ^ k for k in (8, 4, 2, 1)]
        nidx = [jnp.full((16,), n, jnp.int32) for n in range(N)]

        def p_body(p, c2):
            ri = p // 8
            cb = (p % 8) * 16
            accs = [seq_rows_v[slot, p, pl.ds(d * 16, 16)] for d in range(4)]
            rel_vec = rel_v[it * NC + ri, pl.ds(cb, 16)]
            w_vec = jnp.where(rel_vec == 0, w_r[0],
                              jnp.where(rel_vec == 1, w_r[1], w_r[2]))
            for n in range(N):
                w_n = w_vec.at[nidx[n]].get(mode="promise_in_bounds")
                row = p * N + n
                for d in range(4):
                    accs[d] = accs[d] + w_n * nbr_rows_v[slot, row,
                                                         pl.ds(d * 16, 16)]
            ssv = accs[0] * accs[0]
            for d in range(1, 4):
                ssv = ssv + accs[d] * accs[d]
            for ix in bidx:
                ssv = ssv + ssv.at[ix].get(mode="promise_in_bounds")
            x = jnp.maximum(ssv, jnp.float32(1e-24))
            yi = jnp.int32(0x5F3759DF) - lax.shift_right_logical(
                lax.bitcast_convert_type(x, jnp.int32), 1)
            y = lax.bitcast_convert_type(yi, jnp.float32)
            for _ in range(3):
                y = y * (1.5 - 0.5 * x * y * y)
            for d in range(4):
                out_v[slot, p, pl.ds(d * 16, 16)] = accs[d] * y
            return c2

        lax.fori_loop(0, P, p_body, 0)
        pltpu.async_copy(out_v.at[slot], out_hbm.at[pl.ds(base, P)],
                         sem_out[slot])

    fire(0, 0)

    def g_body(g, carry):
        it = g * 2
        fire(it + 1, 1)
        compute(it, 0)

        @pl.when(it + 2 < NIT)
        def _():
            fire(it + 2, 0)

        compute(it + 1, 1)
        return carry

    lax.fori_loop(0, NIT // 2, g_body, 0)
    drain_out(0)
    drain_out(1)


@jax.jit
def _run(seq_flat, nbr2d, rel2d, uem, wb_pad, r_pad):
    uem = jnp.pad(uem, ((0, 0), (0, DPAD - DIM)))
    mesh = plsc.VectorSubcoreMesh(core_axis_name="c", subcore_axis_name="s")
    k = functools.partial(
        pl.kernel,
        mesh=mesh,
        out_type=jax.ShapeDtypeStruct((BL, DPAD), jnp.float32),
        scratch_types=[
            pltpu.VMEM((PW,), jnp.int32),
            pltpu.VMEM((WR, 128), jnp.int32),
            pltpu.VMEM((WR, 128), jnp.int32),
            pltpu.VMEM((128,), jnp.float32),
            pltpu.VMEM((8, 128), jnp.float32),
            pltpu.VMEM((2, P, DPAD), jnp.float32),
            pltpu.VMEM((2, P * N, DPAD), jnp.float32),
            pltpu.VMEM((2, P, DPAD), jnp.float32),
            pltpu.SemaphoreType.DMA,
            pltpu.SemaphoreType.DMA,
            pltpu.SemaphoreType.DMA,
            pltpu.SemaphoreType.DMA,
        ],
    )(_body)
    return k(seq_flat, nbr2d, rel2d, uem, wb_pad, r_pad)


def kernel(sequence, seq_neighbor, rel_neigh, UEm, R, weight_b):
    seq_flat = sequence.reshape(BL)
    nbr2d = seq_neighbor.reshape(NROWS, 128)
    rel2d = rel_neigh.reshape(NROWS, 128)
    wb_pad = jnp.full((128,), -1e30, jnp.float32).at[:RELA_NUM].set(weight_b[:, 0])
    r_pad = jnp.zeros((8, 128), jnp.float32).at[:RELA_NUM, :RELA_NUM].set(R)
    out = _run(seq_flat, nbr2d, rel2d, UEm, wb_pad, r_pad)
    return out[:, :DIM].reshape(B, L, DIM)


# speedup vs baseline: 1.1153x; 1.0020x over previous
import functools

import jax
import jax.numpy as jnp
from jax import lax
from jax.experimental import pallas as pl
from jax.experimental.pallas import tpu as pltpu
from jax.experimental.pallas import tpu_sc as plsc

DIM = 64
DPAD = 128
RELA_NUM = 3
B, L, N = 1024, 50, 16
BL = B * L
NW = 32
PW = BL // NW
P = 16
NIT = PW // P
NC = P * N // 128
NROWS = BL * N // 128
WR = NROWS // NW


def _body(seq_hbm, nbr_hbm, rel_hbm, tab_hbm, wb_hbm, r_hbm, out_hbm,
          seq_idx_v, nbr_idx_v, rel_v, wb_v, r_v,
          seq_rows_v, nbr_rows_v, out_v, sem_in0, sem_in1, sem_out0, sem_out1):
    wid = lax.axis_index("s") * 2 + lax.axis_index("c")
    sem_in = (sem_in0, sem_in1)
    sem_out = (sem_out0, sem_out1)

    pltpu.sync_copy(wb_hbm, wb_v)
    pltpu.sync_copy(r_hbm, r_v)
    wb = wb_v[pl.ds(0, 16)]
    m = jnp.maximum(jnp.maximum(wb[0], wb[1]), wb[2])
    e = jnp.exp(wb - m)
    beta = e / (e[0] + e[1] + e[2])
    w_r = []
    for r in range(RELA_NUM):
        t = r_v[r, pl.ds(0, 16)] * beta
        w_r.append(t[0] + t[1] + t[2])

    pltpu.sync_copy(seq_hbm.at[pl.ds(wid * PW, PW)], seq_idx_v)
    pltpu.sync_copy(nbr_hbm.at[pl.ds(wid * WR, WR)], nbr_idx_v)
    pltpu.sync_copy(rel_hbm.at[pl.ds(wid * WR, WR)], rel_v)

    def fire(it, slot):
        pltpu.async_copy(tab_hbm.at[seq_idx_v.at[pl.ds(it * P, P)]],
                         seq_rows_v.at[slot], sem_in[slot])
        for c in range(NC):
            pltpu.async_copy(tab_hbm.at[nbr_idx_v.at[it * NC + c]],
                             nbr_rows_v.at[slot, pl.ds(c * 128, 128)],
                             sem_in[slot])

    def wait_rows(it, slot):
        pltpu.make_async_copy(tab_hbm.at[seq_idx_v.at[pl.ds(it * P, P)]],
                              seq_rows_v.at[slot], sem_in[slot]).wait()
        for c in range(NC):
            pltpu.make_async_copy(tab_hbm.at[nbr_idx_v.at[it * NC + c]],
                                  nbr_rows_v.at[slot, pl.ds(c * 128, 128)],
                                  sem_in[slot]).wait()

    def drain_out(slot):
        pltpu.make_async_copy(out_v.at[slot],
                              out_hbm.at[pl.ds(0, P)], sem_out[slot]).wait()

    def compute(it, slot):
        base = wid * PW + it * P
        wait_rows(it, slot)

        @pl.when(it >= 2)
        def _():
            drain_out(slot)

        lane = lax.iota(jnp.int32, 16)
        bidx = [lane ^ k for k in (8, 4, 2, 1)]
        nidx = [jnp.full((16,), n, jnp.int32) for n in range(N)]

        def p_body(p, c2):
            ri = p // 8
            cb = (p % 8) * 16
            accs = [seq_rows_v[slot, p, pl.ds(d * 16, 16)] for d in range(4)]
            rel_vec = rel_v[it * NC + ri, pl.ds(cb, 16)]
            w_vec = jnp.where(rel_vec == 0, w_r[0],
                              jnp.where(rel_vec == 1, w_r[1], w_r[2]))
            for n in range(N):
                w_n = w_vec.at[nidx[n]].get(mode="promise_in_bounds")
                row = p * N + n
                for d in range(4):
                    accs[d] = accs[d] + w_n * nbr_rows_v[slot, row,
                                                         pl.ds(d * 16, 16)]
            ssv = accs[0] * accs[0]
            for d in range(1, 4):
                ssv = ssv + accs[d] * accs[d]
            for ix in bidx:
                ssv = ssv + ssv.at[ix].get(mode="promise_in_bounds")
            x = jnp.maximum(ssv, jnp.float32(1e-24))
            yi = jnp.int32(0x5F3759DF) - lax.shift_right_logical(
                lax.bitcast_convert_type(x, jnp.int32), 1)
            y = lax.bitcast_convert_type(yi, jnp.float32)
            for _ in range(3):
                y = y * (1.5 - 0.5 * x * y * y)
            for d in range(4):
                out_v[slot, p, pl.ds(d * 16, 16)] = accs[d] * y
            return c2

        lax.fori_loop(0, P, p_body, 0)
        pltpu.async_copy(out_v.at[slot], out_hbm.at[pl.ds(base, P)],
                         sem_out[slot])

    fire(0, 0)

    def g_body(g, carry):
        it = g * 2
        fire(it + 1, 1)
        compute(it, 0)

        @pl.when(it + 2 < NIT)
        def _():
            fire(it + 2, 0)

        compute(it + 1, 1)
        return carry

    lax.fori_loop(0, NIT // 2, g_body, 0)
    drain_out(0)
    drain_out(1)


@jax.jit
def _run(seq_flat, nbr2d, rel2d, uem, wb_pad, r_pad):
    mesh = plsc.VectorSubcoreMesh(core_axis_name="c", subcore_axis_name="s")
    k = functools.partial(
        pl.kernel,
        mesh=mesh,
        out_type=jax.ShapeDtypeStruct((BL, DPAD), jnp.float32),
        scratch_types=[
            pltpu.VMEM((PW,), jnp.int32),
            pltpu.VMEM((WR, 128), jnp.int32),
            pltpu.VMEM((WR, 128), jnp.int32),
            pltpu.VMEM((128,), jnp.float32),
            pltpu.VMEM((8, 128), jnp.float32),
            pltpu.VMEM((2, P, DPAD), jnp.float32),
            pltpu.VMEM((2, P * N, DPAD), jnp.float32),
            pltpu.VMEM((2, P, DPAD), jnp.float32),
            pltpu.SemaphoreType.DMA,
            pltpu.SemaphoreType.DMA,
            pltpu.SemaphoreType.DMA,
            pltpu.SemaphoreType.DMA,
        ],
    )(_body)
    return k(seq_flat, nbr2d, rel2d, uem, wb_pad, r_pad)


def kernel(sequence, seq_neighbor, rel_neigh, UEm, R, weight_b):
    UEm = jnp.pad(UEm, ((0, 0), (0, DPAD - DIM)))
    seq_flat = sequence.reshape(BL)
    nbr2d = seq_neighbor.reshape(NROWS, 128)
    rel2d = rel_neigh.reshape(NROWS, 128)
    wb_pad = jnp.full((128,), -1e30, jnp.float32).at[:RELA_NUM].set(weight_b[:, 0])
    r_pad = jnp.zeros((8, 128), jnp.float32).at[:RELA_NUM, :RELA_NUM].set(R)
    out = _run(seq_flat, nbr2d, rel2d, UEm, wb_pad, r_pad)
    return out[:, :DIM].reshape(B, L, DIM)
